# baseline (speedup 1.0000x reference)
import functools

import jax
import jax.numpy as jnp
from jax import lax
from jax.experimental import pallas as pl
from jax.experimental.pallas import tpu as pltpu
from jax.experimental.pallas import tpu_sc as plsc


def _proj_body(x_ref, w_ref, wb_ref, u_ref, v_ref):
    p = jnp.dot(w_ref[...], x_ref[0], preferred_element_type=jnp.float32)
    e = jnp.exp(-(p + wb_ref[...]))
    u_ref[0] = e[0:1, :]
    v_ref[0] = e[1:2, :]


def _make_sc_pairwise(BS, N):
    L = 16
    NW = 32
    RPW = (BS * N) // NW
    WPB = N // RPW
    G = 16
    NG = RPW // G
    CH = N // L
    seg_chunks = RPW // L

    mesh = plsc.VectorSubcoreMesh(core_axis_name="c", subcore_axis_name="s")

    @functools.partial(
        pl.kernel,
        out_type=jax.ShapeDtypeStruct((BS * N, N), jnp.float32),
        mesh=mesh,
        scratch_types=[
            pltpu.VMEM((N,), jnp.float32),
            pltpu.VMEM((N,), jnp.float32),
            pltpu.VMEM((G, N), jnp.float32),
            pltpu.VMEM((G, N), jnp.float32),
            pltpu.SemaphoreType.DMA,
            pltpu.SemaphoreType.DMA,
        ],
    )
    def sc_pair(u_hbm, v_hbm, out_hbm, u_v, v_v, buf0, buf1, sem0, sem1):
        cid = lax.axis_index("c")
        sid = lax.axis_index("s")
        wid = sid * 2 + cid
        bw = wid // WPB
        seg = wid % WPB
        base_row = wid * RPW
        cg0 = seg * seg_chunks

        pltpu.sync_copy(u_hbm.at[bw], u_v)
        pltpu.sync_copy(v_hbm.at[bw], v_v)

        jl = lax.iota(jnp.int32, 16)

        def emit_group(g, buf):
            cg = cg0 + g
            uc = u_v[pl.ds(cg * L, L)]
            vc = v_v[pl.ds(cg * L, L)]
            u_spl = [jnp.full((L,), uc[r], dtype=jnp.float32) for r in range(G)]
            v_spl = [jnp.full((L,), vc[r], dtype=jnp.float32) for r in range(G)]

            @plsc.parallel_loop(0, cg, unroll=2)
            def _lower(c):
                uch = u_v[pl.ds(c * L, L)]
                for r in range(G):
                    buf[r, pl.ds(c * L, L)] = 1.0 / (1.0 + uch * v_spl[r])

            @plsc.parallel_loop(cg + 1, CH, unroll=2)
            def _upper(c):
                vch = v_v[pl.ds(c * L, L)]
                for r in range(G):
                    buf[r, pl.ds(c * L, L)] = 1.0 / (1.0 + u_spl[r] * vch)

            for r in range(G):
                e = jnp.where(jl > r, u_spl[r] * vc, uc * v_spl[r])
                s = 1.0 / (1.0 + e)
                buf[r, pl.ds(cg * L, L)] = jnp.where(jl == r, 0.0, s)

        def loop_body(gg, _):
            g0 = gg * 2
            g1 = gg * 2 + 1

            @pl.when(gg > 0)
            def _():
                pltpu.make_async_copy(
                    buf0, out_hbm.at[pl.ds(base_row, G)], sem0).wait()

            emit_group(g0, buf0)
            pltpu.async_copy(
                buf0, out_hbm.at[pl.ds(base_row + g0 * G, G)], sem0)

            @pl.when(gg > 0)
            def _():
                pltpu.make_async_copy(
                    buf1, out_hbm.at[pl.ds(base_row, G)], sem1).wait()

            emit_group(g1, buf1)
            pltpu.async_copy(
                buf1, out_hbm.at[pl.ds(base_row + g1 * G, G)], sem1)
            return 0

        lax.fori_loop(0, NG // 2, loop_body, 0)
        pltpu.make_async_copy(buf0, out_hbm.at[pl.ds(base_row, G)], sem0).wait()
        pltpu.make_async_copy(buf1, out_hbm.at[pl.ds(base_row, G)], sem1).wait()

    return sc_pair


def _pair_body(u_ref, v_ref, out_ref, *, tile_i, n):
    t = pl.program_id(1)
    i0 = t * tile_i
    u_full = u_ref[0, 0, :]
    v_full = v_ref[0, 0, :]
    u_rows = u_ref[0, 0, pl.ds(i0, tile_i)]
    v_rows = v_ref[0, 0, pl.ds(i0, tile_i)]

    ii = i0 + jax.lax.broadcasted_iota(jnp.int32, (tile_i, n), 0)
    jj = jax.lax.broadcasted_iota(jnp.int32, (tile_i, n), 1)

    e = jnp.where(ii < jj,
                  u_rows[:, None] * v_full[None, :],
                  u_full[None, :] * v_rows[:, None])
    s = 1.0 / (1.0 + e)
    out_ref[0] = jnp.where(ii == jj, 0.0, s)


def kernel(node_features, W, b):
    B, F, N = node_features.shape
    Wr = W.reshape(2, F)
    Wb = jnp.stack([jnp.zeros((), W.dtype), b[0]]).reshape(2, 1)

    u3, v3 = pl.pallas_call(
        _proj_body,
        grid=(B,),
        in_specs=[
            pl.BlockSpec((1, F, N), lambda i: (i, 0, 0)),
            pl.BlockSpec((2, F), lambda i: (0, 0)),
            pl.BlockSpec((2, 1), lambda i: (0, 0)),
        ],
        out_specs=[
            pl.BlockSpec((1, 1, N), lambda i: (i, 0, 0)),
            pl.BlockSpec((1, 1, N), lambda i: (i, 0, 0)),
        ],
        out_shape=[
            jax.ShapeDtypeStruct((B, 1, N), jnp.float32),
            jax.ShapeDtypeStruct((B, 1, N), jnp.float32),
        ],
    )(node_features, Wr, Wb)

    BS = 8
    u = u3.reshape(B, N)
    v = v3.reshape(B, N)
    sc_out = _make_sc_pairwise(BS, N)(u[:BS], v[:BS]).reshape(BS, N, N)

    TI = 256
    body = functools.partial(_pair_body, tile_i=TI, n=N)
    tc_out = pl.pallas_call(
        body,
        grid=(B - BS, N // TI),
        in_specs=[
            pl.BlockSpec((1, 1, N), lambda i, t: (BS + i, 0, 0)),
            pl.BlockSpec((1, 1, N), lambda i, t: (BS + i, 0, 0)),
        ],
        out_specs=pl.BlockSpec((1, TI, N), lambda i, t: (i, t, 0)),
        out_shape=jax.ShapeDtypeStruct((B - BS, N, N), jnp.float32),
    )(u3, v3)

    return jnp.concatenate([sc_out, tc_out], axis=0)


# speedup vs baseline: 1.5205x; 1.5205x over previous
import functools

import jax
import jax.numpy as jnp
from jax import lax
from jax.experimental import pallas as pl
from jax.experimental.pallas import tpu as pltpu
from jax.experimental.pallas import tpu_sc as plsc


def _proj_body(x_ref, w_ref, wb_ref, u_ref, v_ref):
    p = jnp.dot(w_ref[...], x_ref[0], preferred_element_type=jnp.float32)
    e = jnp.exp(-(p + wb_ref[...]))
    u_ref[0] = e[0:1, :]
    v_ref[0] = e[1:2, :]


def _make_sc_pairwise(B, N):
    L = 16
    NW = 32
    RPW = (B * N) // NW
    G = 16
    NG = RPW // G
    CH = N // L
    half_chunks = RPW // L

    mesh = plsc.VectorSubcoreMesh(core_axis_name="c", subcore_axis_name="s")

    @functools.partial(
        pl.kernel,
        out_type=jax.ShapeDtypeStruct((B * N, N), jnp.float32),
        mesh=mesh,
        scratch_types=[
            pltpu.VMEM((N,), jnp.float32),
            pltpu.VMEM((N,), jnp.float32),
            pltpu.VMEM((G, N), jnp.float32),
            pltpu.VMEM((G, N), jnp.float32),
            pltpu.SemaphoreType.DMA,
            pltpu.SemaphoreType.DMA,
        ],
    )
    def sc_pair(u_hbm, v_hbm, out_hbm, u_v, v_v, buf0, buf1, sem0, sem1):
        cid = lax.axis_index("c")
        sid = lax.axis_index("s")
        wid = sid * 2 + cid
        bw = wid // 2
        half = wid % 2
        base_row = wid * RPW
        cg0 = half * half_chunks

        pltpu.sync_copy(u_hbm.at[bw], u_v)
        pltpu.sync_copy(v_hbm.at[bw], v_v)

        jl = lax.iota(jnp.int32, 16)

        def emit_group(g, buf):
            cg = cg0 + g
            uc = u_v[pl.ds(cg * L, L)]
            vc = v_v[pl.ds(cg * L, L)]
            u_spl = [jnp.full((L,), uc[r], dtype=jnp.float32) for r in range(G)]
            v_spl = [jnp.full((L,), vc[r], dtype=jnp.float32) for r in range(G)]

            @plsc.parallel_loop(0, cg, unroll=2)
            def _lower(c):
                uch = u_v[pl.ds(c * L, L)]
                for r in range(G):
                    buf[r, pl.ds(c * L, L)] = 1.0 / (1.0 + uch * v_spl[r])

            @plsc.parallel_loop(cg + 1, CH, unroll=2)
            def _upper(c):
                vch = v_v[pl.ds(c * L, L)]
                for r in range(G):
                    buf[r, pl.ds(c * L, L)] = 1.0 / (1.0 + u_spl[r] * vch)

            for r in range(G):
                e = jnp.where(jl > r, u_spl[r] * vc, uc * v_spl[r])
                s = 1.0 / (1.0 + e)
                buf[r, pl.ds(cg * L, L)] = jnp.where(jl == r, 0.0, s)

        def loop_body(gg, _):
            g0 = gg * 2
            g1 = gg * 2 + 1

            @pl.when(gg > 0)
            def _():
                pltpu.make_async_copy(
                    buf0, out_hbm.at[pl.ds(base_row, G)], sem0).wait()

            emit_group(g0, buf0)
            pltpu.async_copy(
                buf0, out_hbm.at[pl.ds(base_row + g0 * G, G)], sem0)

            @pl.when(gg > 0)
            def _():
                pltpu.make_async_copy(
                    buf1, out_hbm.at[pl.ds(base_row, G)], sem1).wait()

            emit_group(g1, buf1)
            pltpu.async_copy(
                buf1, out_hbm.at[pl.ds(base_row + g1 * G, G)], sem1)
            return 0

        lax.fori_loop(0, NG // 2, loop_body, 0)
        pltpu.make_async_copy(buf0, out_hbm.at[pl.ds(base_row, G)], sem0).wait()
        pltpu.make_async_copy(buf1, out_hbm.at[pl.ds(base_row, G)], sem1).wait()

    return sc_pair


def kernel(node_features, W, b):
    B, F, N = node_features.shape
    Wr = W.reshape(2, F)
    Wb = jnp.stack([jnp.zeros((), W.dtype), b[0]]).reshape(2, 1)

    NS = N // 2
    u3, v3 = pl.pallas_call(
        _proj_body,
        grid=(B, 2),
        in_specs=[
            pl.BlockSpec((1, F, NS), lambda i, j: (i, 0, j)),
            pl.BlockSpec((2, F), lambda i, j: (0, 0)),
            pl.BlockSpec((2, 1), lambda i, j: (0, 0)),
        ],
        out_specs=[
            pl.BlockSpec((1, 1, NS), lambda i, j: (i, 0, j)),
            pl.BlockSpec((1, 1, NS), lambda i, j: (i, 0, j)),
        ],
        out_shape=[
            jax.ShapeDtypeStruct((B, 1, N), jnp.float32),
            jax.ShapeDtypeStruct((B, 1, N), jnp.float32),
        ],
    )(node_features, Wr, Wb)

    out_flat = _make_sc_pairwise(B, N)(u3.reshape(B, N), v3.reshape(B, N))
    return out_flat.reshape(B, N, N)


# speedup vs baseline: 1.7310x; 1.1385x over previous
import functools

import jax
import jax.numpy as jnp
from jax import lax
from jax.experimental import pallas as pl
from jax.experimental.pallas import tpu as pltpu
from jax.experimental.pallas import tpu_sc as plsc


def _proj_body(x_ref, w_ref, wb_ref, u_ref, v_ref):
    p = jnp.dot(w_ref[...], x_ref[0], preferred_element_type=jnp.float32)
    e = jnp.exp(-(p + wb_ref[...]))
    u_ref[0] = e[0:1, :]
    v_ref[0] = e[1:2, :]


def _make_sc_pairwise(B, N):
    L = 16
    NW = 32
    RPW = (B * N) // NW
    G = 16
    NG = RPW // G
    CH = N // L
    half_chunks = RPW // L

    mesh = plsc.VectorSubcoreMesh(core_axis_name="c", subcore_axis_name="s")

    @functools.partial(
        pl.kernel,
        out_type=jax.ShapeDtypeStruct((B * N, N), jnp.float32),
        mesh=mesh,
        scratch_types=[
            pltpu.VMEM((N,), jnp.float32),
            pltpu.VMEM((N,), jnp.float32),
            pltpu.VMEM((G, N), jnp.float32),
            pltpu.VMEM((G, N), jnp.float32),
            pltpu.SemaphoreType.DMA,
            pltpu.SemaphoreType.DMA,
        ],
    )
    def sc_pair(u_hbm, v_hbm, out_hbm, u_v, v_v, buf0, buf1, sem0, sem1):
        cid = lax.axis_index("c")
        sid = lax.axis_index("s")
        wid = sid * 2 + cid
        bw = wid // 2
        half = wid % 2
        base_row = wid * RPW
        cg0 = half * half_chunks

        pltpu.sync_copy(u_hbm.at[bw], u_v)
        pltpu.sync_copy(v_hbm.at[bw], v_v)

        jl = lax.iota(jnp.int32, 16)

        def emit_group(g, buf):
            cg = cg0 + g
            uc = u_v[pl.ds(cg * L, L)]
            vc = v_v[pl.ds(cg * L, L)]
            u_spl = [jnp.full((L,), uc[r], dtype=jnp.float32) for r in range(G)]
            v_spl = [jnp.full((L,), vc[r], dtype=jnp.float32) for r in range(G)]

            @plsc.parallel_loop(0, cg, unroll=2)
            def _lower(c):
                uch = u_v[pl.ds(c * L, L)]
                for r in range(G):
                    buf[r, pl.ds(c * L, L)] = 1.0 / (1.0 + uch * v_spl[r])

            @plsc.parallel_loop(cg + 1, CH, unroll=2)
            def _upper(c):
                vch = v_v[pl.ds(c * L, L)]
                for r in range(G):
                    buf[r, pl.ds(c * L, L)] = 1.0 / (1.0 + u_spl[r] * vch)

            for r in range(G):
                e = jnp.where(jl > r, u_spl[r] * vc, uc * v_spl[r])
                s = 1.0 / (1.0 + e)
                buf[r, pl.ds(cg * L, L)] = jnp.where(jl == r, 0.0, s)

        def loop_body(gg, _):
            g0 = gg * 2
            g1 = gg * 2 + 1

            @pl.when(gg > 0)
            def _():
                pltpu.make_async_copy(
                    buf0, out_hbm.at[pl.ds(base_row, G)], sem0).wait()

            emit_group(g0, buf0)
            pltpu.async_copy(
                buf0, out_hbm.at[pl.ds(base_row + g0 * G, G)], sem0)

            @pl.when(gg > 0)
            def _():
                pltpu.make_async_copy(
                    buf1, out_hbm.at[pl.ds(base_row, G)], sem1).wait()

            emit_group(g1, buf1)
            pltpu.async_copy(
                buf1, out_hbm.at[pl.ds(base_row + g1 * G, G)], sem1)
            return 0

        lax.fori_loop(0, NG // 2, loop_body, 0)
        pltpu.make_async_copy(buf0, out_hbm.at[pl.ds(base_row, G)], sem0).wait()
        pltpu.make_async_copy(buf1, out_hbm.at[pl.ds(base_row, G)], sem1).wait()

    return sc_pair


def kernel(node_features, W, b):
    B, F, N = node_features.shape
    Wr = W.reshape(2, F)
    Wb = jnp.stack([jnp.zeros((), W.dtype), b[0]]).reshape(2, 1)

    u3, v3 = pl.pallas_call(
        _proj_body,
        grid=(B,),
        in_specs=[
            pl.BlockSpec((1, F, N), lambda i: (i, 0, 0)),
            pl.BlockSpec((2, F), lambda i: (0, 0)),
            pl.BlockSpec((2, 1), lambda i: (0, 0)),
        ],
        out_specs=[
            pl.BlockSpec((1, 1, N), lambda i: (i, 0, 0)),
            pl.BlockSpec((1, 1, N), lambda i: (i, 0, 0)),
        ],
        out_shape=[
            jax.ShapeDtypeStruct((B, 1, N), jnp.float32),
            jax.ShapeDtypeStruct((B, 1, N), jnp.float32),
        ],
    )(node_features, Wr, Wb)

    out_flat = _make_sc_pairwise(B, N)(u3.reshape(B, N), v3.reshape(B, N))
    return out_flat.reshape(B, N, N)


# speedup vs baseline: 1.9221x; 1.1104x over previous
import functools

import jax
import jax.numpy as jnp
from jax import lax
from jax.experimental import pallas as pl
from jax.experimental.pallas import tpu as pltpu
from jax.experimental.pallas import tpu_sc as plsc


def _proj_body(x_ref, w_ref, wb_ref, u_ref, v_ref):
    for k in range(4):
        p = jnp.dot(w_ref[...], x_ref[k], preferred_element_type=jnp.float32)
        e = jnp.exp(-(p + wb_ref[...]))
        u_ref[k] = e[0:1, :]
        v_ref[k] = e[1:2, :]


def _make_sc_pairwise(B, N):
    L = 16
    NW = 32
    RPW = (B * N) // NW
    G = 16
    NG = RPW // G
    CH = N // L
    half_chunks = RPW // L

    mesh = plsc.VectorSubcoreMesh(core_axis_name="c", subcore_axis_name="s")

    @functools.partial(
        pl.kernel,
        out_type=jax.ShapeDtypeStruct((B * N, N), jnp.float32),
        mesh=mesh,
        scratch_types=[
            pltpu.VMEM((N,), jnp.float32),
            pltpu.VMEM((N,), jnp.float32),
            pltpu.VMEM((G, N), jnp.float32),
            pltpu.VMEM((G, N), jnp.float32),
            pltpu.SemaphoreType.DMA,
            pltpu.SemaphoreType.DMA,
        ],
    )
    def sc_pair(u_hbm, v_hbm, out_hbm, u_v, v_v, buf0, buf1, sem0, sem1):
        cid = lax.axis_index("c")
        sid = lax.axis_index("s")
        wid = sid * 2 + cid
        bw = wid // 2
        half = wid % 2
        base_row = wid * RPW
        cg0 = half * half_chunks

        pltpu.sync_copy(u_hbm.at[bw], u_v)
        pltpu.sync_copy(v_hbm.at[bw], v_v)

        jl = lax.iota(jnp.int32, 16)

        def emit_group(g, buf):
            cg = cg0 + g
            uc = u_v[pl.ds(cg * L, L)]
            vc = v_v[pl.ds(cg * L, L)]
            u_spl = [jnp.full((L,), uc[r], dtype=jnp.float32) for r in range(G)]
            v_spl = [jnp.full((L,), vc[r], dtype=jnp.float32) for r in range(G)]

            @plsc.parallel_loop(0, cg, unroll=2)
            def _lower(c):
                uch = u_v[pl.ds(c * L, L)]
                for r in range(G):
                    buf[r, pl.ds(c * L, L)] = 1.0 / (1.0 + uch * v_spl[r])

            @plsc.parallel_loop(cg + 1, CH, unroll=2)
            def _upper(c):
                vch = v_v[pl.ds(c * L, L)]
                for r in range(G):
                    buf[r, pl.ds(c * L, L)] = 1.0 / (1.0 + u_spl[r] * vch)

            for r in range(G):
                e = jnp.where(jl > r, u_spl[r] * vc, uc * v_spl[r])
                s = 1.0 / (1.0 + e)
                buf[r, pl.ds(cg * L, L)] = jnp.where(jl == r, 0.0, s)

        def loop_body(gg, _):
            g0 = gg * 2
            g1 = gg * 2 + 1

            @pl.when(gg > 0)
            def _():
                pltpu.make_async_copy(
                    buf0, out_hbm.at[pl.ds(base_row, G)], sem0).wait()

            emit_group(g0, buf0)
            pltpu.async_copy(
                buf0, out_hbm.at[pl.ds(base_row + g0 * G, G)], sem0)

            @pl.when(gg > 0)
            def _():
                pltpu.make_async_copy(
                    buf1, out_hbm.at[pl.ds(base_row, G)], sem1).wait()

            emit_group(g1, buf1)
            pltpu.async_copy(
                buf1, out_hbm.at[pl.ds(base_row + g1 * G, G)], sem1)
            return 0

        lax.fori_loop(0, NG // 2, loop_body, 0)
        pltpu.make_async_copy(buf0, out_hbm.at[pl.ds(base_row, G)], sem0).wait()
        pltpu.make_async_copy(buf1, out_hbm.at[pl.ds(base_row, G)], sem1).wait()

    return sc_pair


def kernel(node_features, W, b):
    B, F, N = node_features.shape
    Wr = W.reshape(2, F)
    Wb = jnp.stack([jnp.zeros((), W.dtype), b[0]]).reshape(2, 1)

    u3, v3 = pl.pallas_call(
        _proj_body,
        grid=(B // 4,),
        in_specs=[
            pl.BlockSpec((4, F, N), lambda i: (i, 0, 0)),
            pl.BlockSpec((2, F), lambda i: (0, 0)),
            pl.BlockSpec((2, 1), lambda i: (0, 0)),
        ],
        out_specs=[
            pl.BlockSpec((4, 1, N), lambda i: (i, 0, 0)),
            pl.BlockSpec((4, 1, N), lambda i: (i, 0, 0)),
        ],
        out_shape=[
            jax.ShapeDtypeStruct((B, 1, N), jnp.float32),
            jax.ShapeDtypeStruct((B, 1, N), jnp.float32),
        ],
    )(node_features, Wr, Wb)

    out_flat = _make_sc_pairwise(B, N)(u3.reshape(B, N), v3.reshape(B, N))
    return out_flat.reshape(B, N, N)


# speedup vs baseline: 1.9308x; 1.0045x over previous
import functools

import jax
import jax.numpy as jnp
from jax import lax
from jax.experimental import pallas as pl
from jax.experimental.pallas import tpu as pltpu
from jax.experimental.pallas import tpu_sc as plsc


def _proj_body(x_ref, w_ref, wb_ref, u_ref, v_ref):
    for k in range(8):
        p = jnp.dot(w_ref[...], x_ref[k], preferred_element_type=jnp.float32)
        e = jnp.exp(-(p + wb_ref[...]))
        u_ref[k] = e[0:1, :]
        v_ref[k] = e[1:2, :]


def _make_sc_pairwise(B, N):
    L = 16
    NW = 32
    RPW = (B * N) // NW
    G = 16
    NG = RPW // G
    CH = N // L
    half_chunks = RPW // L

    mesh = plsc.VectorSubcoreMesh(core_axis_name="c", subcore_axis_name="s")

    @functools.partial(
        pl.kernel,
        out_type=jax.ShapeDtypeStruct((B * N, N), jnp.float32),
        mesh=mesh,
        scratch_types=[
            pltpu.VMEM((N,), jnp.float32),
            pltpu.VMEM((N,), jnp.float32),
            pltpu.VMEM((G, N), jnp.float32),
            pltpu.VMEM((G, N), jnp.float32),
            pltpu.SemaphoreType.DMA,
            pltpu.SemaphoreType.DMA,
        ],
    )
    def sc_pair(u_hbm, v_hbm, out_hbm, u_v, v_v, buf0, buf1, sem0, sem1):
        cid = lax.axis_index("c")
        sid = lax.axis_index("s")
        wid = sid * 2 + cid
        bw = wid // 2
        half = wid % 2
        base_row = wid * RPW
        cg0 = half * half_chunks

        pltpu.sync_copy(u_hbm.at[bw], u_v)
        pltpu.sync_copy(v_hbm.at[bw], v_v)

        jl = lax.iota(jnp.int32, 16)

        def emit_group(g, buf):
            cg = cg0 + g
            uc = u_v[pl.ds(cg * L, L)]
            vc = v_v[pl.ds(cg * L, L)]
            u_spl = [jnp.full((L,), uc[r], dtype=jnp.float32) for r in range(G)]
            v_spl = [jnp.full((L,), vc[r], dtype=jnp.float32) for r in range(G)]

            @plsc.parallel_loop(0, cg, unroll=2)
            def _lower(c):
                uch = u_v[pl.ds(c * L, L)]
                for r in range(G):
                    buf[r, pl.ds(c * L, L)] = 1.0 / (1.0 + uch * v_spl[r])

            @plsc.parallel_loop(cg + 1, CH, unroll=2)
            def _upper(c):
                vch = v_v[pl.ds(c * L, L)]
                for r in range(G):
                    buf[r, pl.ds(c * L, L)] = 1.0 / (1.0 + u_spl[r] * vch)

            for r in range(G):
                e = jnp.where(jl > r, u_spl[r] * vc, uc * v_spl[r])
                s = 1.0 / (1.0 + e)
                buf[r, pl.ds(cg * L, L)] = jnp.where(jl == r, 0.0, s)

        def loop_body(gg, _):
            g0 = gg * 2
            g1 = gg * 2 + 1

            @pl.when(gg > 0)
            def _():
                pltpu.make_async_copy(
                    buf0, out_hbm.at[pl.ds(base_row, G)], sem0).wait()

            emit_group(g0, buf0)
            pltpu.async_copy(
                buf0, out_hbm.at[pl.ds(base_row + g0 * G, G)], sem0)

            @pl.when(gg > 0)
            def _():
                pltpu.make_async_copy(
                    buf1, out_hbm.at[pl.ds(base_row, G)], sem1).wait()

            emit_group(g1, buf1)
            pltpu.async_copy(
                buf1, out_hbm.at[pl.ds(base_row + g1 * G, G)], sem1)
            return 0

        lax.fori_loop(0, NG // 2, loop_body, 0)
        pltpu.make_async_copy(buf0, out_hbm.at[pl.ds(base_row, G)], sem0).wait()
        pltpu.make_async_copy(buf1, out_hbm.at[pl.ds(base_row, G)], sem1).wait()

    return sc_pair


def kernel(node_features, W, b):
    B, F, N = node_features.shape
    Wr = W.reshape(2, F)
    Wb = jnp.stack([jnp.zeros((), W.dtype), b[0]]).reshape(2, 1)

    u3, v3 = pl.pallas_call(
        _proj_body,
        grid=(B // 8,),
        in_specs=[
            pl.BlockSpec((8, F, N), lambda i: (i, 0, 0)),
            pl.BlockSpec((2, F), lambda i: (0, 0)),
            pl.BlockSpec((2, 1), lambda i: (0, 0)),
        ],
        out_specs=[
            pl.BlockSpec((8, 1, N), lambda i: (i, 0, 0)),
            pl.BlockSpec((8, 1, N), lambda i: (i, 0, 0)),
        ],
        out_shape=[
            jax.ShapeDtypeStruct((B, 1, N), jnp.float32),
            jax.ShapeDtypeStruct((B, 1, N), jnp.float32),
        ],
    )(node_features, Wr, Wb)

    out_flat = _make_sc_pairwise(B, N)(u3.reshape(B, N), v3.reshape(B, N))
    return out_flat.reshape(B, N, N)
